# baseline (speedup 1.0000x reference)
import functools

import jax
import jax.numpy as jnp
from jax import lax
from jax.experimental import pallas as pl
from jax.experimental.pallas import tpu as pltpu
from jax.experimental.pallas import tpu_sc as plsc

N_P = 50000
N_D = 50000
E = 500000
D = 128
OUT = 3
L = 3

NPAD = 51200
TRASH = 50000
CW = 32
NCHUNK = 4
KB = 3
KE = KB * 128
NBLK = 82
TPT = KE * NBLK
EPAD = 16 * TPT
STRIPE = NPAD // 16
ZR = 320
ZRC = 400

_f32 = jnp.float32


def _make_agg_body(srcs):
    nrel = len(srcs)

    def body(*refs):
        xs = {'p': refs[0], 'd': refs[1]}
        edges = refs[2:2 + 2 * nrel]
        zrows = refs[2 + 2 * nrel]
        outs = refs[3 + 2 * nrel: 3 + 3 * nrel]
        idx_g, idx_s, rows, acc, sem_g, sem_i = refs[3 + 3 * nrel:]
        core = lax.axis_index("c")
        sub = lax.axis_index("s")

        for r in range(nrel):
            xstk = xs[srcs[r]]
            src3d = edges[2 * r]
            dst2d = edges[2 * r + 1]

            @pl.loop(jnp.int32(0), jnp.int32(2))
            def _chunk(cc, xstk=xstk, src3d=src3d, dst2d=dst2d, out=outs[r]):
                c = core * jnp.int32(2) + cc

                def load_idx_sync(sl, blk_no):
                    base = sub * (KB * NBLK) + blk_no * KB
                    pltpu.sync_copy(src3d.at[c, pl.ds(base, KB)],
                                    idx_g.at[jnp.int32(sl)])
                    pltpu.sync_copy(dst2d.at[pl.ds(base, KB)],
                                    idx_s.at[jnp.int32(sl)])

                def load_idx_async(sl, blk_no):
                    base = sub * (KB * NBLK) + blk_no * KB
                    pltpu.async_copy(src3d.at[c, pl.ds(base, KB)],
                                     idx_g.at[jnp.int32(sl)], sem_i)
                    pltpu.async_copy(dst2d.at[pl.ds(base, KB)],
                                     idx_s.at[jnp.int32(sl)], sem_i)

                def wait_idx(sl):
                    pltpu.make_async_copy(src3d.at[jnp.int32(0), pl.ds(0, KB)],
                                          idx_g.at[jnp.int32(sl)], sem_i).wait()
                    pltpu.make_async_copy(dst2d.at[pl.ds(0, KB)],
                                          idx_s.at[jnp.int32(sl)], sem_i).wait()

                def fire_gathers(sl):
                    for j in range(KB):
                        pltpu.async_copy(
                            xstk.at[idx_g.at[jnp.int32(sl), jnp.int32(j)]],
                            rows.at[jnp.int32(sl), pl.ds(j * 128, 128)], sem_g)

                def wait_gathers(sl):
                    for j in range(KB):
                        pltpu.make_async_copy(
                            xstk.at[idx_g.at[jnp.int32(sl), jnp.int32(j)]],
                            rows.at[jnp.int32(sl), pl.ds(j * 128, 128)],
                            sem_g).wait()

                def scatter(sl):
                    for j in range(KB):
                        pltpu.sync_copy(
                            rows.at[jnp.int32(sl), pl.ds(j * 128, 128)],
                            acc.at[idx_s.at[jnp.int32(sl), jnp.int32(j)]],
                            add=True)

                pltpu.sync_copy(zrows, rows.at[jnp.int32(0), pl.ds(0, ZR)])
                for i in range(STRIPE // ZR):
                    pltpu.sync_copy(rows.at[jnp.int32(0), pl.ds(0, ZR)],
                                    acc.at[pl.ds(sub * STRIPE + i * ZR, ZR)])
                plsc.subcore_barrier()

                load_idx_sync(0, jnp.int32(0))
                fire_gathers(0)
                load_idx_async(1, jnp.int32(1))

                @pl.loop(jnp.int32(0), jnp.int32(NBLK), step=2)
                def blk(b0):
                    for p in range(2):
                        b = b0 + p
                        wait_gathers(p)

                        @pl.when(b + 1 < NBLK)
                        def _fire_next():
                            wait_idx(1 - p)
                            fire_gathers(1 - p)
                        scatter(p)

                        @pl.when(b + 2 < NBLK)
                        def _prefetch_idx():
                            load_idx_async(p, b + 2)
                plsc.subcore_barrier()
                for i in range(STRIPE // ZR):
                    off = sub * STRIPE + i * ZR
                    pltpu.sync_copy(acc.at[pl.ds(off, ZR)],
                                    rows.at[jnp.int32(0), pl.ds(0, ZR)])
                    pltpu.sync_copy(rows.at[jnp.int32(0), pl.ds(0, ZR)],
                                    out.at[pl.ds(c * jnp.int32(NPAD) + off, ZR)])
    return body


@functools.lru_cache(maxsize=None)
def _get_sc_agg(srcs):
    nrel = len(srcs)
    return pl.kernel(
        _make_agg_body(srcs),
        out_type=[jax.ShapeDtypeStruct((NCHUNK * NPAD, CW), _f32)
                  for _ in range(nrel)],
        mesh=plsc.VectorSubcoreMesh(core_axis_name="c", subcore_axis_name="s",
                                    num_cores=2, num_subcores=16),
        compiler_params=pltpu.CompilerParams(use_tc_tiling_on_sc=False),
        scratch_types=[
            pltpu.VMEM((2, KB, 128), jnp.int32),
            pltpu.VMEM((2, KB, 128), jnp.int32),
            pltpu.VMEM((2, KE, CW), _f32),
            pltpu.VMEM_SHARED((NPAD, CW), _f32),
            pltpu.SemaphoreType.DMA,
            pltpu.SemaphoreType.DMA,
        ],
    )


def _sc_counts_body(d0, d1, d2, d3, z8, ones8, c0, c1, c2, c3,
                    idx_s, ones_v, zbuf, stage, acc):
    core = lax.axis_index("c")
    sub = lax.axis_index("s")
    dsts = (d0, d1, d2, d3)
    outs = (c0, c1, c2, c3)
    pltpu.sync_copy(z8, zbuf)
    pltpu.sync_copy(ones8, ones_v)
    for r in range(NCHUNK):
        @pl.when(core == (r // 2))
        def _task(r=r):
            for i in range(STRIPE // ZRC):
                pltpu.sync_copy(zbuf,
                                acc.at[pl.ds(sub * STRIPE + i * ZRC, ZRC)])
            plsc.subcore_barrier()

            @pl.loop(jnp.int32(0), jnp.int32(NBLK))
            def blk(b):
                base = sub * (KB * NBLK) + b * KB
                pltpu.sync_copy(dsts[r].at[pl.ds(base, KB)], idx_s)
                for j in range(KB):
                    pltpu.sync_copy(ones_v, acc.at[idx_s.at[jnp.int32(j)]], add=True)
            plsc.subcore_barrier()
            for i in range(STRIPE // ZRC):
                off = sub * STRIPE + i * ZRC
                pltpu.sync_copy(acc.at[pl.ds(off, ZRC)], stage)
                pltpu.sync_copy(stage, outs[r].at[pl.ds(off, ZRC)])


BN = 1024


def _combine_body(sa0, sa1, sa2, sa3, sb0, sb1, sb2, sb3,
                  xc0, xc1, xc2, xc3, ca, cb,
                  wa, wb, wra, wrb, bia, bib, g, be,
                  y0, y1, y2, y3):
    sa = jnp.concatenate([sa0[...], sa1[...], sa2[...], sa3[...]], axis=1)
    sb = jnp.concatenate([sb0[...], sb1[...], sb2[...], sb3[...]], axis=1)
    x = jnp.concatenate([xc0[...], xc1[...], xc2[...], xc3[...]], axis=1)
    inva = 1.0 / jnp.maximum(ca[...][:, 0:1], 1.0)
    invb = 1.0 / jnp.maximum(cb[...][:, 0:1], 1.0)
    h = (jnp.dot(sa * inva, wa[...], preferred_element_type=_f32)
         + jnp.dot(sb * invb, wb[...], preferred_element_type=_f32)
         + jnp.dot(x, wra[...] + wrb[...], preferred_element_type=_f32)
         + bia[...] + bib[...])
    m = jnp.mean(h, axis=1, keepdims=True)
    v = jnp.mean((h - m) * (h - m), axis=1, keepdims=True)
    hn = (h - m) * lax.rsqrt(v + 1e-5) * g[...] + be[...]
    y = jnp.maximum(hn, 0.0)
    y0[...] = y[:, 0:32]
    y1[...] = y[:, 32:64]
    y2[...] = y[:, 64:96]
    y3[...] = y[:, 96:128]


def _final_body(sa0, sa1, sa2, sa3, sb0, sb1, sb2, sb3,
                xc0, xc1, xc2, xc3, ca, cb,
                wa, wb, wra, wrb, bia, bib, g, be, wo, bo, out):
    sa = jnp.concatenate([sa0[...], sa1[...], sa2[...], sa3[...]], axis=1)
    sb = jnp.concatenate([sb0[...], sb1[...], sb2[...], sb3[...]], axis=1)
    x = jnp.concatenate([xc0[...], xc1[...], xc2[...], xc3[...]], axis=1)
    inva = 1.0 / jnp.maximum(ca[...][:, 0:1], 1.0)
    invb = 1.0 / jnp.maximum(cb[...][:, 0:1], 1.0)
    h = (jnp.dot(sa * inva, wa[...], preferred_element_type=_f32)
         + jnp.dot(sb * invb, wb[...], preferred_element_type=_f32)
         + jnp.dot(x, wra[...] + wrb[...], preferred_element_type=_f32)
         + bia[...] + bib[...])
    m = jnp.mean(h, axis=1, keepdims=True)
    v = jnp.mean((h - m) * (h - m), axis=1, keepdims=True)
    hn = (h - m) * lax.rsqrt(v + 1e-5) * g[...] + be[...]
    y = jnp.maximum(hn, 0.0)
    out[...] = jnp.dot(y, wo[...], preferred_element_type=_f32) + bo[...]


def _chunk_spec():
    return pl.BlockSpec((BN, CW), lambda i: (i, jnp.int32(0)))


def _cnt_spec():
    return pl.BlockSpec((BN, 8), lambda i: (i, jnp.int32(0)))


def _w_spec():
    return pl.BlockSpec((D, D), lambda i: (jnp.int32(0), jnp.int32(0)))


def _b_spec():
    return pl.BlockSpec((1, D), lambda i: (jnp.int32(0), jnp.int32(0)))


_combine = pl.pallas_call(
    _combine_body,
    grid=(NPAD // BN,),
    in_specs=[_chunk_spec() for _ in range(12)]
    + [_cnt_spec(), _cnt_spec()]
    + [_w_spec() for _ in range(4)]
    + [_b_spec() for _ in range(4)],
    out_specs=[_chunk_spec() for _ in range(4)],
    out_shape=[jax.ShapeDtypeStruct((NPAD, CW), _f32) for _ in range(4)],
)

_final = pl.pallas_call(
    _final_body,
    grid=(NPAD // BN,),
    in_specs=[_chunk_spec() for _ in range(12)]
    + [_cnt_spec(), _cnt_spec()]
    + [_w_spec() for _ in range(4)]
    + [_b_spec() for _ in range(4)]
    + [pl.BlockSpec((D, 8), lambda i: (jnp.int32(0), jnp.int32(0))), pl.BlockSpec((1, 8), lambda i: (jnp.int32(0), jnp.int32(0)))],
    out_specs=pl.BlockSpec((BN, 8), lambda i: (i, jnp.int32(0))),
    out_shape=jax.ShapeDtypeStruct((NPAD, 8), _f32),
)

@functools.lru_cache(maxsize=None)
def _get_sc_counts():
    return pl.kernel(
        _sc_counts_body,
        out_type=[jax.ShapeDtypeStruct((NPAD, 8), _f32) for _ in range(4)],
        mesh=plsc.VectorSubcoreMesh(core_axis_name="c", subcore_axis_name="s",
                                    num_cores=2, num_subcores=16),
        compiler_params=pltpu.CompilerParams(use_tc_tiling_on_sc=False),
        scratch_types=[
            pltpu.VMEM((KB, 128), jnp.int32),
            pltpu.VMEM((128, 8), _f32),
            pltpu.VMEM((ZRC, 8), _f32),
            pltpu.VMEM((ZRC, 8), _f32),
            pltpu.VMEM_SHARED((NPAD, 8), _f32),
        ],
    )


def _prep_edges(ei):
    src = ei[0].astype(jnp.int32)
    dst = ei[1].astype(jnp.int32)
    pad = EPAD - E
    src = jnp.concatenate([src, jnp.zeros((pad,), jnp.int32)])
    dst = jnp.concatenate([dst, jnp.full((pad,), TRASH, jnp.int32)])
    src2d = src.reshape(EPAD // 128, 128)
    offs = (jnp.arange(NCHUNK, dtype=jnp.int32) * NPAD)[:, None, None]
    return src2d[None, :, :] + offs, dst.reshape(EPAD // 128, 128)


def _chunks(x, n):
    xp = jnp.zeros((NPAD, D), _f32).at[:n].set(x.astype(_f32))
    return [xp[:, c * CW:(c + 1) * CW] for c in range(NCHUNK)]


def kernel(x_primal, x_dual, ei_p2p, ei_d2d, ei_p2d, ei_d2p,
           Wrel, brel, Wroot, ln_gp, ln_bp, ln_gd, ln_bd, W_out, b_out):
    Wrel = Wrel.astype(_f32)
    brel = brel.astype(_f32)
    Wroot = Wroot.astype(_f32)

    e_pp = _prep_edges(ei_p2p)
    e_dd = _prep_edges(ei_d2d)
    e_pd = _prep_edges(ei_p2d)
    e_dp = _prep_edges(ei_d2p)

    zrows = jnp.zeros((ZR, CW), _f32)

    xp = _chunks(x_primal, N_P)
    xd = _chunks(x_dual, N_D)

    z8 = jnp.zeros((ZRC, 8), _f32)
    ones8 = jnp.ones((128, 8), _f32)
    cnts = _get_sc_counts()(e_pp[1], e_dd[1], e_pd[1], e_dp[1], z8, ones8)
    c_pp, c_dd, c_pd, c_dp = cnts

    def combine(fn, Sa, Sb, xc, ca, cb, l, ra, rb, g, be, extra=()):
        args = (list(Sa) + list(Sb) + list(xc)
                + [ca, cb,
                   Wrel[l, ra], Wrel[l, rb], Wroot[l, ra], Wroot[l, rb],
                   brel[l, ra][None, :], brel[l, rb][None, :],
                   g[l][None, :], be[l][None, :]]
                + list(extra))
        return fn(*args)

    aggp = _get_sc_agg(('p',))
    aggd = _get_sc_agg(('d',))

    def unstk(so):
        return [so[c * NPAD:(c + 1) * NPAD] for c in range(NCHUNK)]

    for l in range(L - 1):
        xps = jnp.concatenate(xp, axis=0)
        xds = jnp.concatenate(xd, axis=0)
        s_pp = unstk(aggp(xps, xds, *e_pp, zrows)[0])
        s_dp = unstk(aggd(xps, xds, *e_dp, zrows)[0])
        s_dd = unstk(aggd(xps, xds, *e_dd, zrows)[0])
        s_pd = unstk(aggp(xps, xds, *e_pd, zrows)[0])
        xp_new = combine(_combine, s_pp, s_dp, xp, c_pp, c_dp, l, 0, 3, ln_gp, ln_bp)
        xd_new = combine(_combine, s_dd, s_pd, xd, c_dd, c_pd, l, 1, 2, ln_gd, ln_bd)
        xp, xd = list(xp_new), list(xd_new)

    xps = jnp.concatenate(xp, axis=0)
    xds = jnp.concatenate(xd, axis=0)
    s_pp = unstk(aggp(xps, xds, *e_pp, zrows)[0])
    s_dp = unstk(aggd(xps, xds, *e_dp, zrows)[0])
    wo = jnp.zeros((D, 8), _f32).at[:, :OUT].set(W_out.astype(_f32))
    bo = jnp.zeros((1, 8), _f32).at[0, :OUT].set(b_out.astype(_f32))
    out8 = combine(_final, s_pp, s_dp, xp, c_pp, c_dp, L - 1, 0, 3,
                   ln_gp, ln_bp, extra=(wo, bo))
    return out8[:N_P, :OUT].astype(x_primal.dtype)


# speedup vs baseline: 1.2734x; 1.2734x over previous
import functools

import jax
import jax.numpy as jnp
from jax import lax
from jax.experimental import pallas as pl
from jax.experimental.pallas import tpu as pltpu
from jax.experimental.pallas import tpu_sc as plsc

N_P = 50000
N_D = 50000
E = 500000
D = 128
OUT = 3
L = 3

NPAD = 51200
TRASH = 50000
CW = 32
NCHUNK = 4
KB = 3
KE = KB * 128
NBLK = 82
TPT = KE * NBLK
EPAD = 16 * TPT
STRIPE = NPAD // 16
ZR = 320
ZRC = 400

_f32 = jnp.float32


def _make_agg_body(srcs):
    nrel = len(srcs)

    def body(*refs):
        xs = {'p': refs[0:4], 'd': refs[4:8]}
        edges = refs[8:8 + 2 * nrel]
        zrows = refs[8 + 2 * nrel]
        outs = refs[9 + 2 * nrel: 9 + 6 * nrel]
        idx_g, idx_s, rows, acc, sem_g, sem_i, sem_o = refs[9 + 6 * nrel:]
        core = lax.axis_index("c")
        sub = lax.axis_index("s")

        for r in range(nrel):
            xc4 = xs[srcs[r]]
            src2d = edges[2 * r]
            dst2d = edges[2 * r + 1]
            for c in range(NCHUNK):
                @pl.when(core == (c // 2))
                def _task(r=r, c=c, xc=xc4[c], src2d=src2d, dst2d=dst2d):
                    def load_idx_sync(sl, blk_no):
                        base = sub * (KB * NBLK) + blk_no * KB
                        pltpu.sync_copy(src2d.at[pl.ds(base, KB)],
                                        idx_g.at[jnp.int32(sl)])
                        pltpu.sync_copy(dst2d.at[pl.ds(base, KB)],
                                        idx_s.at[jnp.int32(sl)])

                    def load_idx_async(sl, blk_no):
                        base = sub * (KB * NBLK) + blk_no * KB
                        pltpu.async_copy(src2d.at[pl.ds(base, KB)],
                                         idx_g.at[jnp.int32(sl)], sem_i)
                        pltpu.async_copy(dst2d.at[pl.ds(base, KB)],
                                         idx_s.at[jnp.int32(sl)], sem_i)

                    def wait_idx(sl):
                        pltpu.make_async_copy(src2d.at[pl.ds(0, KB)],
                                              idx_g.at[jnp.int32(sl)], sem_i).wait()
                        pltpu.make_async_copy(dst2d.at[pl.ds(0, KB)],
                                              idx_s.at[jnp.int32(sl)], sem_i).wait()

                    def fire_gathers(sl):
                        for j in range(KB):
                            pltpu.async_copy(
                                xc.at[idx_g.at[jnp.int32(sl), jnp.int32(j)]],
                                rows.at[jnp.int32(sl), pl.ds(j * 128, 128)], sem_g)

                    def wait_gathers(sl):
                        for j in range(KB):
                            pltpu.make_async_copy(
                                xc.at[idx_g.at[jnp.int32(sl), jnp.int32(j)]],
                                rows.at[jnp.int32(sl), pl.ds(j * 128, 128)],
                                sem_g).wait()

                    def scatter(sl):
                        for j in range(KB):
                            pltpu.sync_copy(
                                rows.at[jnp.int32(sl), pl.ds(j * 128, 128)],
                                acc.at[idx_s.at[jnp.int32(sl), jnp.int32(j)]],
                                add=True)

                    pltpu.sync_copy(zrows, rows.at[jnp.int32(0), pl.ds(0, ZR)])
                    for i in range(STRIPE // ZR):
                        pltpu.sync_copy(rows.at[jnp.int32(0), pl.ds(0, ZR)],
                                        acc.at[pl.ds(sub * STRIPE + i * ZR, ZR)])
                    plsc.subcore_barrier()

                    load_idx_sync(0, jnp.int32(0))
                    fire_gathers(0)
                    load_idx_async(1, jnp.int32(1))

                    @pl.loop(jnp.int32(0), jnp.int32(NBLK), step=2)
                    def blk(b0):
                        for p in range(2):
                            b = b0 + p
                            wait_gathers(p)

                            @pl.when(b + 1 < NBLK)
                            def _fire_next():
                                wait_idx(1 - p)
                                fire_gathers(1 - p)
                            scatter(p)

                            @pl.when(b + 2 < NBLK)
                            def _prefetch_idx():
                                load_idx_async(p, b + 2)
                    plsc.subcore_barrier()
                    out = outs[r * NCHUNK + c]
                    for i in range(STRIPE // ZR):
                        sl = jnp.int32(i % 2)
                        off = sub * STRIPE + i * ZR
                        if i >= 2:
                            poff = sub * STRIPE + (i - 2) * ZR
                            pltpu.make_async_copy(
                                rows.at[sl, pl.ds(0, ZR)],
                                out.at[pl.ds(poff, ZR)], sem_o).wait()
                        pltpu.sync_copy(acc.at[pl.ds(off, ZR)],
                                        rows.at[sl, pl.ds(0, ZR)])
                        pltpu.async_copy(rows.at[sl, pl.ds(0, ZR)],
                                         out.at[pl.ds(off, ZR)], sem_o)
                    for i in (STRIPE // ZR - 2, STRIPE // ZR - 1):
                        sl = jnp.int32(i % 2)
                        off = sub * STRIPE + i * ZR
                        pltpu.make_async_copy(rows.at[sl, pl.ds(0, ZR)],
                                              out.at[pl.ds(off, ZR)],
                                              sem_o).wait()
    return body


@functools.lru_cache(maxsize=None)
def _get_sc_agg(srcs):
    nrel = len(srcs)
    return pl.kernel(
        _make_agg_body(srcs),
        out_type=[jax.ShapeDtypeStruct((NPAD, CW), _f32)
                  for _ in range(NCHUNK * nrel)],
        mesh=plsc.VectorSubcoreMesh(core_axis_name="c", subcore_axis_name="s",
                                    num_cores=2, num_subcores=16),
        compiler_params=pltpu.CompilerParams(use_tc_tiling_on_sc=False),
        scratch_types=[
            pltpu.VMEM((2, KB, 128), jnp.int32),
            pltpu.VMEM((2, KB, 128), jnp.int32),
            pltpu.VMEM((2, KE, CW), _f32),
            pltpu.VMEM_SHARED((NPAD, CW), _f32),
            pltpu.SemaphoreType.DMA,
            pltpu.SemaphoreType.DMA,
            pltpu.SemaphoreType.DMA,
        ],
    )


def _sc_counts_body(d0, d1, d2, d3, z8, ones8, c0, c1, c2, c3,
                    idx_s, ones_v, zbuf, stage, acc):
    core = lax.axis_index("c")
    sub = lax.axis_index("s")
    dsts = (d0, d1, d2, d3)
    outs = (c0, c1, c2, c3)
    pltpu.sync_copy(z8, zbuf)
    pltpu.sync_copy(ones8, ones_v)
    for r in range(NCHUNK):
        @pl.when(core == (r // 2))
        def _task(r=r):
            for i in range(STRIPE // ZRC):
                pltpu.sync_copy(zbuf,
                                acc.at[pl.ds(sub * STRIPE + i * ZRC, ZRC)])
            plsc.subcore_barrier()

            @pl.loop(jnp.int32(0), jnp.int32(NBLK))
            def blk(b):
                base = sub * (KB * NBLK) + b * KB
                pltpu.sync_copy(dsts[r].at[pl.ds(base, KB)], idx_s)
                for j in range(KB):
                    pltpu.sync_copy(ones_v, acc.at[idx_s.at[jnp.int32(j)]], add=True)
            plsc.subcore_barrier()
            for i in range(STRIPE // ZRC):
                off = sub * STRIPE + i * ZRC
                pltpu.sync_copy(acc.at[pl.ds(off, ZRC)], stage)
                pltpu.sync_copy(stage, outs[r].at[pl.ds(off, ZRC)])


BN = 1024


def _combine_body(sa0, sa1, sa2, sa3, sb0, sb1, sb2, sb3,
                  xc0, xc1, xc2, xc3, ca, cb,
                  wa, wb, wra, wrb, bia, bib, g, be,
                  y0, y1, y2, y3):
    sa = jnp.concatenate([sa0[...], sa1[...], sa2[...], sa3[...]], axis=1)
    sb = jnp.concatenate([sb0[...], sb1[...], sb2[...], sb3[...]], axis=1)
    x = jnp.concatenate([xc0[...], xc1[...], xc2[...], xc3[...]], axis=1)
    inva = 1.0 / jnp.maximum(ca[...][:, 0:1], 1.0)
    invb = 1.0 / jnp.maximum(cb[...][:, 0:1], 1.0)
    h = (jnp.dot(sa * inva, wa[...], preferred_element_type=_f32)
         + jnp.dot(sb * invb, wb[...], preferred_element_type=_f32)
         + jnp.dot(x, wra[...] + wrb[...], preferred_element_type=_f32)
         + bia[...] + bib[...])
    m = jnp.mean(h, axis=1, keepdims=True)
    v = jnp.mean((h - m) * (h - m), axis=1, keepdims=True)
    hn = (h - m) * lax.rsqrt(v + 1e-5) * g[...] + be[...]
    y = jnp.maximum(hn, 0.0)
    y0[...] = y[:, 0:32]
    y1[...] = y[:, 32:64]
    y2[...] = y[:, 64:96]
    y3[...] = y[:, 96:128]


def _final_body(sa0, sa1, sa2, sa3, sb0, sb1, sb2, sb3,
                xc0, xc1, xc2, xc3, ca, cb,
                wa, wb, wra, wrb, bia, bib, g, be, wo, bo, out):
    sa = jnp.concatenate([sa0[...], sa1[...], sa2[...], sa3[...]], axis=1)
    sb = jnp.concatenate([sb0[...], sb1[...], sb2[...], sb3[...]], axis=1)
    x = jnp.concatenate([xc0[...], xc1[...], xc2[...], xc3[...]], axis=1)
    inva = 1.0 / jnp.maximum(ca[...][:, 0:1], 1.0)
    invb = 1.0 / jnp.maximum(cb[...][:, 0:1], 1.0)
    h = (jnp.dot(sa * inva, wa[...], preferred_element_type=_f32)
         + jnp.dot(sb * invb, wb[...], preferred_element_type=_f32)
         + jnp.dot(x, wra[...] + wrb[...], preferred_element_type=_f32)
         + bia[...] + bib[...])
    m = jnp.mean(h, axis=1, keepdims=True)
    v = jnp.mean((h - m) * (h - m), axis=1, keepdims=True)
    hn = (h - m) * lax.rsqrt(v + 1e-5) * g[...] + be[...]
    y = jnp.maximum(hn, 0.0)
    out[...] = jnp.dot(y, wo[...], preferred_element_type=_f32) + bo[...]


def _chunk_spec():
    return pl.BlockSpec((BN, CW), lambda i: (i, jnp.int32(0)))


def _cnt_spec():
    return pl.BlockSpec((BN, 8), lambda i: (i, jnp.int32(0)))


def _w_spec():
    return pl.BlockSpec((D, D), lambda i: (jnp.int32(0), jnp.int32(0)))


def _b_spec():
    return pl.BlockSpec((1, D), lambda i: (jnp.int32(0), jnp.int32(0)))


_combine = pl.pallas_call(
    _combine_body,
    grid=(NPAD // BN,),
    in_specs=[_chunk_spec() for _ in range(12)]
    + [_cnt_spec(), _cnt_spec()]
    + [_w_spec() for _ in range(4)]
    + [_b_spec() for _ in range(4)],
    out_specs=[_chunk_spec() for _ in range(4)],
    out_shape=[jax.ShapeDtypeStruct((NPAD, CW), _f32) for _ in range(4)],
)

_final = pl.pallas_call(
    _final_body,
    grid=(NPAD // BN,),
    in_specs=[_chunk_spec() for _ in range(12)]
    + [_cnt_spec(), _cnt_spec()]
    + [_w_spec() for _ in range(4)]
    + [_b_spec() for _ in range(4)]
    + [pl.BlockSpec((D, 8), lambda i: (jnp.int32(0), jnp.int32(0))), pl.BlockSpec((1, 8), lambda i: (jnp.int32(0), jnp.int32(0)))],
    out_specs=pl.BlockSpec((BN, 8), lambda i: (i, jnp.int32(0))),
    out_shape=jax.ShapeDtypeStruct((NPAD, 8), _f32),
)

@functools.lru_cache(maxsize=None)
def _get_sc_counts():
    return pl.kernel(
        _sc_counts_body,
        out_type=[jax.ShapeDtypeStruct((NPAD, 8), _f32) for _ in range(4)],
        mesh=plsc.VectorSubcoreMesh(core_axis_name="c", subcore_axis_name="s",
                                    num_cores=2, num_subcores=16),
        compiler_params=pltpu.CompilerParams(use_tc_tiling_on_sc=False),
        scratch_types=[
            pltpu.VMEM((KB, 128), jnp.int32),
            pltpu.VMEM((128, 8), _f32),
            pltpu.VMEM((ZRC, 8), _f32),
            pltpu.VMEM((ZRC, 8), _f32),
            pltpu.VMEM_SHARED((NPAD, 8), _f32),
        ],
    )


def _prep_edges(ei):
    src = ei[0].astype(jnp.int32)
    dst = ei[1].astype(jnp.int32)
    pad = EPAD - E
    src = jnp.concatenate([src, jnp.zeros((pad,), jnp.int32)])
    dst = jnp.concatenate([dst, jnp.full((pad,), TRASH, jnp.int32)])
    return src.reshape(EPAD // 128, 128), dst.reshape(EPAD // 128, 128)


def _chunks(x, n):
    xp = jnp.zeros((NPAD, D), _f32).at[:n].set(x.astype(_f32))
    return [xp[:, c * CW:(c + 1) * CW] for c in range(NCHUNK)]


def kernel(x_primal, x_dual, ei_p2p, ei_d2d, ei_p2d, ei_d2p,
           Wrel, brel, Wroot, ln_gp, ln_bp, ln_gd, ln_bd, W_out, b_out):
    Wrel = Wrel.astype(_f32)
    brel = brel.astype(_f32)
    Wroot = Wroot.astype(_f32)

    e_pp = _prep_edges(ei_p2p)
    e_dd = _prep_edges(ei_d2d)
    e_pd = _prep_edges(ei_p2d)
    e_dp = _prep_edges(ei_d2p)

    zrows = jnp.zeros((ZR, CW), _f32)

    xp = _chunks(x_primal, N_P)
    xd = _chunks(x_dual, N_D)

    z8 = jnp.zeros((ZRC, 8), _f32)
    ones8 = jnp.ones((128, 8), _f32)
    cnts = _get_sc_counts()(e_pp[1], e_dd[1], e_pd[1], e_dp[1], z8, ones8)
    c_pp, c_dd, c_pd, c_dp = cnts

    def combine(fn, Sa, Sb, xc, ca, cb, l, ra, rb, g, be, extra=()):
        args = (list(Sa) + list(Sb) + list(xc)
                + [ca, cb,
                   Wrel[l, ra], Wrel[l, rb], Wroot[l, ra], Wroot[l, rb],
                   brel[l, ra][None, :], brel[l, rb][None, :],
                   g[l][None, :], be[l][None, :]]
                + list(extra))
        return fn(*args)

    aggp = _get_sc_agg(('p',))
    aggd = _get_sc_agg(('d',))
    for l in range(L - 1):
        s_pp = aggp(*xp, *xd, *e_pp, zrows)
        s_dp = aggd(*xp, *xd, *e_dp, zrows)
        s_dd = aggd(*xp, *xd, *e_dd, zrows)
        s_pd = aggp(*xp, *xd, *e_pd, zrows)
        xp_new = combine(_combine, s_pp, s_dp, xp, c_pp, c_dp, l, 0, 3, ln_gp, ln_bp)
        xd_new = combine(_combine, s_dd, s_pd, xd, c_dd, c_pd, l, 1, 2, ln_gd, ln_bd)
        xp, xd = list(xp_new), list(xd_new)

    s_pp = aggp(*xp, *xd, *e_pp, zrows)
    s_dp = aggd(*xp, *xd, *e_dp, zrows)
    wo = jnp.zeros((D, 8), _f32).at[:, :OUT].set(W_out.astype(_f32))
    bo = jnp.zeros((1, 8), _f32).at[0, :OUT].set(b_out.astype(_f32))
    out8 = combine(_final, s_pp, s_dp, xp, c_pp, c_dp, L - 1, 0, 3,
                   ln_gp, ln_bp, extra=(wo, bo))
    return out8[:N_P, :OUT].astype(x_primal.dtype)


# speedup vs baseline: 1.2836x; 1.0081x over previous
import functools

import jax
import jax.numpy as jnp
from jax import lax
from jax.experimental import pallas as pl
from jax.experimental.pallas import tpu as pltpu
from jax.experimental.pallas import tpu_sc as plsc

N_P = 50000
N_D = 50000
E = 500000
D = 128
OUT = 3
L = 3

NPAD = 51200
TRASH = 50000
CW = 32
NCHUNK = 4
KB = 3
KE = KB * 128
NBLK = 82
TPT = KE * NBLK
EPAD = 16 * TPT
STRIPE = NPAD // 16
ZR = 320
ZRC = 400
KBC = 6
NBLKC = 41

_f32 = jnp.float32


def _make_agg_body(srcs):
    nrel = len(srcs)

    def body(*refs):
        xs = {'p': refs[0:4], 'd': refs[4:8]}
        edges = refs[8:8 + 2 * nrel]
        zrows = refs[8 + 2 * nrel]
        outs = refs[9 + 2 * nrel: 9 + 6 * nrel]
        idx_g, idx_s, rows, acc, sem_g, sem_i, sem_o = refs[9 + 6 * nrel:]
        core = lax.axis_index("c")
        sub = lax.axis_index("s")

        for r in range(nrel):
            xc4 = xs[srcs[r]]
            src2d = edges[2 * r]
            dst2d = edges[2 * r + 1]
            for c in range(NCHUNK):
                @pl.when(core == (c // 2))
                def _task(r=r, c=c, xc=xc4[c], src2d=src2d, dst2d=dst2d):
                    def load_idx_sync(sl, blk_no):
                        base = sub * (KB * NBLK) + blk_no * KB
                        pltpu.sync_copy(src2d.at[pl.ds(base, KB)],
                                        idx_g.at[jnp.int32(sl)])
                        pltpu.sync_copy(dst2d.at[pl.ds(base, KB)],
                                        idx_s.at[jnp.int32(sl)])

                    def load_idx_async(sl, blk_no):
                        base = sub * (KB * NBLK) + blk_no * KB
                        pltpu.async_copy(src2d.at[pl.ds(base, KB)],
                                         idx_g.at[jnp.int32(sl)], sem_i)
                        pltpu.async_copy(dst2d.at[pl.ds(base, KB)],
                                         idx_s.at[jnp.int32(sl)], sem_i)

                    def wait_idx(sl):
                        pltpu.make_async_copy(src2d.at[pl.ds(0, KB)],
                                              idx_g.at[jnp.int32(sl)], sem_i).wait()
                        pltpu.make_async_copy(dst2d.at[pl.ds(0, KB)],
                                              idx_s.at[jnp.int32(sl)], sem_i).wait()

                    def fire_gathers(sl):
                        for j in range(KB):
                            pltpu.async_copy(
                                xc.at[idx_g.at[jnp.int32(sl), jnp.int32(j)]],
                                rows.at[jnp.int32(sl), pl.ds(j * 128, 128)], sem_g)

                    def wait_gathers(sl):
                        for j in range(KB):
                            pltpu.make_async_copy(
                                xc.at[idx_g.at[jnp.int32(sl), jnp.int32(j)]],
                                rows.at[jnp.int32(sl), pl.ds(j * 128, 128)],
                                sem_g).wait()

                    def scatter(sl):
                        for j in range(KB):
                            pltpu.sync_copy(
                                rows.at[jnp.int32(sl), pl.ds(j * 128, 128)],
                                acc.at[idx_s.at[jnp.int32(sl), jnp.int32(j)]],
                                add=True)

                    pltpu.sync_copy(zrows, rows.at[jnp.int32(0), pl.ds(0, ZR)])
                    for i in range(STRIPE // ZR):
                        pltpu.sync_copy(rows.at[jnp.int32(0), pl.ds(0, ZR)],
                                        acc.at[pl.ds(sub * STRIPE + i * ZR, ZR)])
                    plsc.subcore_barrier()

                    load_idx_sync(0, jnp.int32(0))
                    fire_gathers(0)
                    load_idx_async(1, jnp.int32(1))

                    @pl.loop(jnp.int32(0), jnp.int32(NBLK), step=2)
                    def blk(b0):
                        for p in range(2):
                            b = b0 + p
                            wait_gathers(p)

                            @pl.when(b + 1 < NBLK)
                            def _fire_next():
                                wait_idx(1 - p)
                                fire_gathers(1 - p)
                            scatter(p)

                            @pl.when(b + 2 < NBLK)
                            def _prefetch_idx():
                                load_idx_async(p, b + 2)
                    plsc.subcore_barrier()
                    out = outs[r * NCHUNK + c]
                    for i in range(STRIPE // ZR):
                        sl = jnp.int32(i % 2)
                        off = sub * STRIPE + i * ZR
                        if i >= 2:
                            poff = sub * STRIPE + (i - 2) * ZR
                            pltpu.make_async_copy(
                                rows.at[sl, pl.ds(0, ZR)],
                                out.at[pl.ds(poff, ZR)], sem_o).wait()
                        pltpu.sync_copy(acc.at[pl.ds(off, ZR)],
                                        rows.at[sl, pl.ds(0, ZR)])
                        pltpu.async_copy(rows.at[sl, pl.ds(0, ZR)],
                                         out.at[pl.ds(off, ZR)], sem_o)
                    for i in (STRIPE // ZR - 2, STRIPE // ZR - 1):
                        sl = jnp.int32(i % 2)
                        off = sub * STRIPE + i * ZR
                        pltpu.make_async_copy(rows.at[sl, pl.ds(0, ZR)],
                                              out.at[pl.ds(off, ZR)],
                                              sem_o).wait()
    return body


@functools.lru_cache(maxsize=None)
def _get_sc_agg(srcs):
    nrel = len(srcs)
    return pl.kernel(
        _make_agg_body(srcs),
        out_type=[jax.ShapeDtypeStruct((NPAD, CW), _f32)
                  for _ in range(NCHUNK * nrel)],
        mesh=plsc.VectorSubcoreMesh(core_axis_name="c", subcore_axis_name="s",
                                    num_cores=2, num_subcores=16),
        compiler_params=pltpu.CompilerParams(use_tc_tiling_on_sc=False),
        scratch_types=[
            pltpu.VMEM((2, KB, 128), jnp.int32),
            pltpu.VMEM((2, KB, 128), jnp.int32),
            pltpu.VMEM((2, KE, CW), _f32),
            pltpu.VMEM_SHARED((NPAD, CW), _f32),
            pltpu.SemaphoreType.DMA,
            pltpu.SemaphoreType.DMA,
            pltpu.SemaphoreType.DMA,
        ],
    )


def _sc_counts_body(d0, d1, d2, d3, z8, ones8, c0, c1, c2, c3,
                    idx_s, ones_v, zbuf, stage, acc):
    core = lax.axis_index("c")
    sub = lax.axis_index("s")
    dsts = (d0, d1, d2, d3)
    outs = (c0, c1, c2, c3)
    pltpu.sync_copy(z8, zbuf)
    pltpu.sync_copy(ones8, ones_v)
    for r in range(NCHUNK):
        @pl.when(core == (r // 2))
        def _task(r=r):
            for i in range(STRIPE // ZRC):
                pltpu.sync_copy(zbuf,
                                acc.at[pl.ds(sub * STRIPE + i * ZRC, ZRC)])
            plsc.subcore_barrier()

            @pl.loop(jnp.int32(0), jnp.int32(NBLKC))
            def blk(b):
                base = sub * (KBC * NBLKC) + b * KBC
                pltpu.sync_copy(dsts[r].at[pl.ds(base, KBC)], idx_s)
                for j in range(KBC):
                    pltpu.sync_copy(ones_v, acc.at[idx_s.at[jnp.int32(j)]], add=True)
            plsc.subcore_barrier()
            for i in range(STRIPE // ZRC):
                off = sub * STRIPE + i * ZRC
                pltpu.sync_copy(acc.at[pl.ds(off, ZRC)], stage)
                pltpu.sync_copy(stage, outs[r].at[pl.ds(off, ZRC)])


BN = 1024


def _combine_body(sa0, sa1, sa2, sa3, sb0, sb1, sb2, sb3,
                  xc0, xc1, xc2, xc3, ca, cb,
                  wa, wb, wra, wrb, bia, bib, g, be,
                  y0, y1, y2, y3):
    sa = jnp.concatenate([sa0[...], sa1[...], sa2[...], sa3[...]], axis=1)
    sb = jnp.concatenate([sb0[...], sb1[...], sb2[...], sb3[...]], axis=1)
    x = jnp.concatenate([xc0[...], xc1[...], xc2[...], xc3[...]], axis=1)
    inva = 1.0 / jnp.maximum(ca[...][:, 0:1], 1.0)
    invb = 1.0 / jnp.maximum(cb[...][:, 0:1], 1.0)
    h = (jnp.dot(sa * inva, wa[...], preferred_element_type=_f32)
         + jnp.dot(sb * invb, wb[...], preferred_element_type=_f32)
         + jnp.dot(x, wra[...] + wrb[...], preferred_element_type=_f32)
         + bia[...] + bib[...])
    m = jnp.mean(h, axis=1, keepdims=True)
    v = jnp.mean((h - m) * (h - m), axis=1, keepdims=True)
    hn = (h - m) * lax.rsqrt(v + 1e-5) * g[...] + be[...]
    y = jnp.maximum(hn, 0.0)
    y0[...] = y[:, 0:32]
    y1[...] = y[:, 32:64]
    y2[...] = y[:, 64:96]
    y3[...] = y[:, 96:128]


def _final_body(sa0, sa1, sa2, sa3, sb0, sb1, sb2, sb3,
                xc0, xc1, xc2, xc3, ca, cb,
                wa, wb, wra, wrb, bia, bib, g, be, wo, bo, out):
    sa = jnp.concatenate([sa0[...], sa1[...], sa2[...], sa3[...]], axis=1)
    sb = jnp.concatenate([sb0[...], sb1[...], sb2[...], sb3[...]], axis=1)
    x = jnp.concatenate([xc0[...], xc1[...], xc2[...], xc3[...]], axis=1)
    inva = 1.0 / jnp.maximum(ca[...][:, 0:1], 1.0)
    invb = 1.0 / jnp.maximum(cb[...][:, 0:1], 1.0)
    h = (jnp.dot(sa * inva, wa[...], preferred_element_type=_f32)
         + jnp.dot(sb * invb, wb[...], preferred_element_type=_f32)
         + jnp.dot(x, wra[...] + wrb[...], preferred_element_type=_f32)
         + bia[...] + bib[...])
    m = jnp.mean(h, axis=1, keepdims=True)
    v = jnp.mean((h - m) * (h - m), axis=1, keepdims=True)
    hn = (h - m) * lax.rsqrt(v + 1e-5) * g[...] + be[...]
    y = jnp.maximum(hn, 0.0)
    out[...] = jnp.dot(y, wo[...], preferred_element_type=_f32) + bo[...]


def _chunk_spec():
    return pl.BlockSpec((BN, CW), lambda i: (i, jnp.int32(0)))


def _cnt_spec():
    return pl.BlockSpec((BN, 8), lambda i: (i, jnp.int32(0)))


def _w_spec():
    return pl.BlockSpec((D, D), lambda i: (jnp.int32(0), jnp.int32(0)))


def _b_spec():
    return pl.BlockSpec((1, D), lambda i: (jnp.int32(0), jnp.int32(0)))


_combine = pl.pallas_call(
    _combine_body,
    grid=(NPAD // BN,),
    in_specs=[_chunk_spec() for _ in range(12)]
    + [_cnt_spec(), _cnt_spec()]
    + [_w_spec() for _ in range(4)]
    + [_b_spec() for _ in range(4)],
    out_specs=[_chunk_spec() for _ in range(4)],
    out_shape=[jax.ShapeDtypeStruct((NPAD, CW), _f32) for _ in range(4)],
)

_final = pl.pallas_call(
    _final_body,
    grid=(NPAD // BN,),
    in_specs=[_chunk_spec() for _ in range(12)]
    + [_cnt_spec(), _cnt_spec()]
    + [_w_spec() for _ in range(4)]
    + [_b_spec() for _ in range(4)]
    + [pl.BlockSpec((D, 8), lambda i: (jnp.int32(0), jnp.int32(0))), pl.BlockSpec((1, 8), lambda i: (jnp.int32(0), jnp.int32(0)))],
    out_specs=pl.BlockSpec((BN, 8), lambda i: (i, jnp.int32(0))),
    out_shape=jax.ShapeDtypeStruct((NPAD, 8), _f32),
)

@functools.lru_cache(maxsize=None)
def _get_sc_counts():
    return pl.kernel(
        _sc_counts_body,
        out_type=[jax.ShapeDtypeStruct((NPAD, 8), _f32) for _ in range(4)],
        mesh=plsc.VectorSubcoreMesh(core_axis_name="c", subcore_axis_name="s",
                                    num_cores=2, num_subcores=16),
        compiler_params=pltpu.CompilerParams(use_tc_tiling_on_sc=False),
        scratch_types=[
            pltpu.VMEM((KBC, 128), jnp.int32),
            pltpu.VMEM((128, 8), _f32),
            pltpu.VMEM((ZRC, 8), _f32),
            pltpu.VMEM((ZRC, 8), _f32),
            pltpu.VMEM_SHARED((NPAD, 8), _f32),
        ],
    )


def _prep_edges(ei):
    src = ei[0].astype(jnp.int32)
    dst = ei[1].astype(jnp.int32)
    pad = EPAD - E
    src = jnp.concatenate([src, jnp.zeros((pad,), jnp.int32)])
    dst = jnp.concatenate([dst, jnp.full((pad,), TRASH, jnp.int32)])
    return src.reshape(EPAD // 128, 128), dst.reshape(EPAD // 128, 128)


def _chunks(x, n):
    xp = jnp.zeros((NPAD, D), _f32).at[:n].set(x.astype(_f32))
    return [xp[:, c * CW:(c + 1) * CW] for c in range(NCHUNK)]


def kernel(x_primal, x_dual, ei_p2p, ei_d2d, ei_p2d, ei_d2p,
           Wrel, brel, Wroot, ln_gp, ln_bp, ln_gd, ln_bd, W_out, b_out):
    Wrel = Wrel.astype(_f32)
    brel = brel.astype(_f32)
    Wroot = Wroot.astype(_f32)

    e_pp = _prep_edges(ei_p2p)
    e_dd = _prep_edges(ei_d2d)
    e_pd = _prep_edges(ei_p2d)
    e_dp = _prep_edges(ei_d2p)

    zrows = jnp.zeros((ZR, CW), _f32)

    xp = _chunks(x_primal, N_P)
    xd = _chunks(x_dual, N_D)

    z8 = jnp.zeros((ZRC, 8), _f32)
    ones8 = jnp.ones((128, 8), _f32)
    cnts = _get_sc_counts()(e_pp[1], e_dd[1], e_pd[1], e_dp[1], z8, ones8)
    c_pp, c_dd, c_pd, c_dp = cnts

    def combine(fn, Sa, Sb, xc, ca, cb, l, ra, rb, g, be, extra=()):
        args = (list(Sa) + list(Sb) + list(xc)
                + [ca, cb,
                   Wrel[l, ra], Wrel[l, rb], Wroot[l, ra], Wroot[l, rb],
                   brel[l, ra][None, :], brel[l, rb][None, :],
                   g[l][None, :], be[l][None, :]]
                + list(extra))
        return fn(*args)

    aggp = _get_sc_agg(('p',))
    aggd = _get_sc_agg(('d',))
    for l in range(L - 1):
        s_pp = aggp(*xp, *xd, *e_pp, zrows)
        s_dp = aggd(*xp, *xd, *e_dp, zrows)
        s_dd = aggd(*xp, *xd, *e_dd, zrows)
        s_pd = aggp(*xp, *xd, *e_pd, zrows)
        xp_new = combine(_combine, s_pp, s_dp, xp, c_pp, c_dp, l, 0, 3, ln_gp, ln_bp)
        xd_new = combine(_combine, s_dd, s_pd, xd, c_dd, c_pd, l, 1, 2, ln_gd, ln_bd)
        xp, xd = list(xp_new), list(xd_new)

    s_pp = aggp(*xp, *xd, *e_pp, zrows)
    s_dp = aggd(*xp, *xd, *e_dp, zrows)
    wo = jnp.zeros((D, 8), _f32).at[:, :OUT].set(W_out.astype(_f32))
    bo = jnp.zeros((1, 8), _f32).at[0, :OUT].set(b_out.astype(_f32))
    out8 = combine(_final, s_pp, s_dp, xp, c_pp, c_dp, L - 1, 0, 3,
                   ln_gp, ln_bp, extra=(wo, bo))
    return out8[:N_P, :OUT].astype(x_primal.dtype)
